# baseline (speedup 1.0000x reference)
import functools

import jax
import jax.numpy as jnp
from jax.experimental import pallas as pl
from jax.experimental.pallas import tpu as pltpu
from jax.experimental.pallas import tpu_sc as plsc

K = 16

def _sc_gather(table, idx_flat):
    num, c = idx_flat.shape[0], table.shape[1]
    window = 256
    assert num % window == 0
    idx2 = idx_flat.reshape(1, num)
    mesh = plsc.VectorSubcoreMesh(core_axis_name="core",
                                  subcore_axis_name="subcore")

    @functools.partial(
        pl.kernel,
        out_type=jax.ShapeDtypeStruct((num, c), table.dtype),
        mesh=mesh,
    )
    def gather_kernel(x_hbm, i_hbm, o_hbm):
        def body(i_vmem, o_vmem):
            pltpu.sync_copy(x_hbm.at[i_vmem.at[0]], o_vmem)

        pltpu.emit_pipeline(
            body,
            grid=(num // window,),
            in_specs=[pl.BlockSpec((1, window), index_map=lambda i: (0, i))],
            out_specs=[pl.BlockSpec((window, c), index_map=lambda i: (i, 0))],
            core_axis_name=("core", "subcore"),
            dimension_semantics=(pltpu.PARALLEL,),
        )(i_hbm, o_hbm)

    return gather_kernel(table, idx2)


def _knn_body(x_ref, xt_ref, sqc_ref, idx_ref, scr_ref, *, bm, npad, c):
    x = x_ref[...]
    dot = jax.lax.dot_general(x, xt_ref[...], (((1,), (0,)), ((), ())),
                              precision=jax.lax.Precision.DEFAULT)
    sq_i = jnp.sum(x * x, axis=1, keepdims=True)
    scr_ref[...] = sq_i - 2.0 * dot + sqc_ref[...]

    lane = jax.lax.broadcasted_iota(jnp.int32, (bm, 128), 1)
    inf = jnp.float32(jnp.inf)
    states = []
    nstream = 1
    for s in range(nstream):
        states.append([jnp.full((bm, 128), inf, jnp.float32) for _ in range(4)]
                      + [jnp.full((bm, 128), npad, jnp.int32) for _ in range(4)])
    for r in range(npad // 128):
        m1, m2, m3, m4, i1, i2, i3, i4 = states[r % nstream]
        v = scr_ref[:, r * 128:(r + 1) * 128]
        g = lane + r * 128
        c1 = v < m1
        c2 = v < m2
        c3 = v < m3
        c4 = v < m4
        m4 = jnp.where(c4, jnp.where(c3, m3, v), m4)
        i4 = jnp.where(c4, jnp.where(c3, i3, g), i4)
        m3 = jnp.where(c3, jnp.where(c2, m2, v), m3)
        i3 = jnp.where(c3, jnp.where(c2, i2, g), i3)
        m2 = jnp.where(c2, jnp.where(c1, m1, v), m2)
        i2 = jnp.where(c2, jnp.where(c1, i1, g), i2)
        m1 = jnp.where(c1, v, m1)
        i1 = jnp.where(c1, g, i1)
        states[r % nstream] = [m1, m2, m3, m4, i1, i2, i3, i4]
    cv = jnp.concatenate([st[j] for st in states for j in range(4)], axis=1)
    ci = jnp.concatenate([st[j] for st in states for j in range(4, 8)], axis=1)
    cols = []
    for _ in range(K):
        m = jnp.min(cv, axis=1, keepdims=True)
        j = jnp.min(jnp.where(cv == m, ci, npad), axis=1, keepdims=True)
        cols.append(j)
        cv = jnp.where(ci == j, inf, cv)
    idx_ref[...] = jnp.concatenate(cols, axis=1)[None]


def _mlp_body(p_ref, qg_ref, w2_ref, b1_ref, b2_ref, o_ref, *, bm, c):
    p = p_ref[...]
    qg = qg_ref[...].reshape(bm, K, c)
    h1 = jnp.maximum(qg + p[:, None, :] + b1_ref[...], 0.0)
    h2 = jax.lax.dot_general(h1.reshape(bm * K, c), w2_ref[...],
                             (((1,), (0,)), ((), ())),
                             precision=jax.lax.Precision.DEFAULT)
    h2 = jnp.maximum(h2 + b2_ref[...], 0.0)
    o_ref[...] = jnp.mean(h2.reshape(bm, K, c), axis=1)


def _pq_body(x_ref, ab_ref, pq_ref):
    pq_ref[...] = jax.lax.dot_general(x_ref[...], ab_ref[...],
                                      (((1,), (0,)), ((), ())),
                                      precision=jax.lax.Precision.DEFAULT)


def _shard_body(x, W1, b1, b2, W2, *, n, c, npad, bm, bmo, nshard):
    xpad = jnp.zeros((npad, c), x.dtype).at[:n].set(x)
    xt = xpad.T
    sqc = jnp.sum(xpad * xpad, axis=1)[None, :]
    sqc = jnp.where(jax.lax.iota(jnp.int32, npad)[None, :] >= n, 1e12, sqc)
    ab = jnp.concatenate([W1[:c] - W1[c:], W1[c:]], axis=1)

    nloc = npad // nshard
    d = jax.lax.axis_index("d")

    pq = pl.pallas_call(
        _pq_body,
        grid=(npad // 512,),
        in_specs=[pl.BlockSpec((512, c), lambda i: (i, 0)),
                  pl.BlockSpec((c, 2 * c), lambda i: (0, 0))],
        out_specs=pl.BlockSpec((512, 2 * c), lambda i: (i, 0)),
        out_shape=jax.ShapeDtypeStruct((npad, 2 * c), xpad.dtype),
        compiler_params=pltpu.CompilerParams(
            dimension_semantics=("parallel",)),
    )(xpad, ab)
    q_full = pq[:, c:]

    nsub = 4
    nsb = nloc // nsub
    outs = []
    idxs = []
    qgs = []
    for s in range(nsub):
        xs = jax.lax.dynamic_slice(xpad, (d * nloc + s * nsb, 0), (nsb, c))
        idx3 = pl.pallas_call(
            functools.partial(_knn_body, bm=bm, npad=npad, c=c),
            grid=(nsb // bm,),
            in_specs=[
                pl.BlockSpec((bm, c), lambda i: (i, 0)),
                pl.BlockSpec((c, npad), lambda i: (0, 0)),
                pl.BlockSpec((1, npad), lambda i: (0, 0)),
            ],
            out_specs=pl.BlockSpec((1, bm, K), lambda i: (i, 0, 0)),
            out_shape=jax.ShapeDtypeStruct((nsb // bm, bm, K), jnp.int32),
            scratch_shapes=[pltpu.VMEM((bm, npad), jnp.float32)],
            compiler_params=pltpu.CompilerParams(
                dimension_semantics=("parallel",)),
        )(xs, xt, sqc)
        idxs.append(idx3.reshape(nsb * K))
        qgs.append(_sc_gather(q_full, idxs[-1]))

    for s in range(nsub):
        p_loc = jax.lax.dynamic_slice(pq[:, :c], (d * nloc + s * nsb, 0),
                                      (nsb, c))
        out = pl.pallas_call(
            functools.partial(_mlp_body, bm=bmo, c=c),
            grid=(nsb // bmo,),
            in_specs=[
                pl.BlockSpec((bmo, c), lambda i: (i, 0)),
                pl.BlockSpec((bmo * K, c), lambda i: (i, 0)),
                pl.BlockSpec((c, c), lambda i: (0, 0)),
                pl.BlockSpec((1, c), lambda i: (0, 0)),
                pl.BlockSpec((1, c), lambda i: (0, 0)),
            ],
            out_specs=pl.BlockSpec((bmo, c), lambda i: (i, 0)),
            out_shape=jax.ShapeDtypeStruct((nsb, c), xpad.dtype),
            compiler_params=pltpu.CompilerParams(
                dimension_semantics=("parallel",)),
        )(p_loc, qgs[s], W2, b1[None, :], b2[None, :])
        outs.append(out)
    return jnp.concatenate(outs, axis=0)


def kernel(x, W1, b1, W2, b2):
    n, c = x.shape
    bm = 256
    bmo = 512
    npad = ((n + 2047) // 2048) * 2048

    ndev = len(jax.devices())
    nshard = 2 if (ndev >= 2 and npad % (2 * bm) == 0) else 1
    mesh = jax.make_mesh((nshard,), ("d",))
    P = jax.sharding.PartitionSpec
    body = functools.partial(_shard_body, n=n, c=c, npad=npad, bm=bm, bmo=bmo,
                             nshard=nshard)
    ns = functools.partial(jax.sharding.NamedSharding, mesh)
    rep = ns(P())
    args = [jax.reshard(a, rep) for a in (x, W1, b1, b2, W2)]
    out = jax.shard_map(
        body, mesh=mesh,
        in_specs=(P(), P(), P(), P(), P()),
        out_specs=P("d", None),
        check_vma=False,
    )(*args)
    return out[:n]


# speedup vs baseline: 1.0613x; 1.0613x over previous
import functools

import jax
import jax.numpy as jnp
from jax.experimental import pallas as pl
from jax.experimental.pallas import tpu as pltpu
from jax.experimental.pallas import tpu_sc as plsc

K = 16

def _sc_gather(table, idx_flat):
    num, c = idx_flat.shape[0], table.shape[1]
    window = 256
    assert num % window == 0
    idx2 = idx_flat.reshape(1, num)
    mesh = plsc.VectorSubcoreMesh(core_axis_name="core",
                                  subcore_axis_name="subcore")

    @functools.partial(
        pl.kernel,
        out_type=jax.ShapeDtypeStruct((num, c), table.dtype),
        mesh=mesh,
    )
    def gather_kernel(x_hbm, i_hbm, o_hbm):
        def body(i_vmem, o_vmem):
            pltpu.sync_copy(x_hbm.at[i_vmem.at[0]], o_vmem)

        pltpu.emit_pipeline(
            body,
            grid=(num // window,),
            in_specs=[pl.BlockSpec((1, window), index_map=lambda i: (0, i))],
            out_specs=[pl.BlockSpec((window, c), index_map=lambda i: (i, 0))],
            core_axis_name=("core", "subcore"),
            dimension_semantics=(pltpu.PARALLEL,),
        )(i_hbm, o_hbm)

    return gather_kernel(table, idx2)


def _knn_body(x_ref, xt_ref, sqc_ref, idx_ref, scr_ref, *, bm, npad, c):
    x = x_ref[...]
    dot = jax.lax.dot_general(x, xt_ref[...], (((1,), (0,)), ((), ())),
                              precision=jax.lax.Precision.DEFAULT)
    sq_i = jnp.sum(x * x, axis=1, keepdims=True)
    scr_ref[...] = sq_i - 2.0 * dot + sqc_ref[...]

    lane = jax.lax.broadcasted_iota(jnp.int32, (bm, 128), 1)
    inf = jnp.float32(jnp.inf)
    states = []
    nstream = 1
    for s in range(nstream):
        states.append([jnp.full((bm, 128), inf, jnp.float32) for _ in range(4)]
                      + [jnp.full((bm, 128), npad, jnp.int32) for _ in range(4)])
    for r in range(npad // 128):
        m1, m2, m3, m4, i1, i2, i3, i4 = states[r % nstream]
        v = scr_ref[:, r * 128:(r + 1) * 128]
        g = lane + r * 128
        c1 = v < m1
        c2 = v < m2
        c3 = v < m3
        c4 = v < m4
        m4 = jnp.where(c4, jnp.where(c3, m3, v), m4)
        i4 = jnp.where(c4, jnp.where(c3, i3, g), i4)
        m3 = jnp.where(c3, jnp.where(c2, m2, v), m3)
        i3 = jnp.where(c3, jnp.where(c2, i2, g), i3)
        m2 = jnp.where(c2, jnp.where(c1, m1, v), m2)
        i2 = jnp.where(c2, jnp.where(c1, i1, g), i2)
        m1 = jnp.where(c1, v, m1)
        i1 = jnp.where(c1, g, i1)
        states[r % nstream] = [m1, m2, m3, m4, i1, i2, i3, i4]
    cv = jnp.concatenate([st[j] for st in states for j in range(4)], axis=1)
    ci = jnp.concatenate([st[j] for st in states for j in range(4, 8)], axis=1)
    cols = []
    for _ in range(K):
        m = jnp.min(cv, axis=1, keepdims=True)
        j = jnp.min(jnp.where(cv == m, ci, npad), axis=1, keepdims=True)
        cols.append(j)
        cv = jnp.where(ci == j, inf, cv)
    idx_ref[...] = jnp.concatenate(cols, axis=1)[None]


def _mlp_body(p_ref, qg_ref, w2_ref, b1_ref, b2_ref, o_ref, *, bm, c):
    p = p_ref[...]
    qg = qg_ref[...].reshape(bm, K, c)
    h1 = jnp.maximum(qg + p[:, None, :] + b1_ref[...], 0.0)
    h2 = jax.lax.dot_general(h1.reshape(bm * K, c), w2_ref[...],
                             (((1,), (0,)), ((), ())),
                             precision=jax.lax.Precision.DEFAULT)
    h2 = jnp.maximum(h2 + b2_ref[...], 0.0)
    o_ref[...] = jnp.mean(h2.reshape(bm, K, c), axis=1)


def _pq_body(x_ref, ab_ref, pq_ref):
    pq_ref[...] = jax.lax.dot_general(x_ref[...], ab_ref[...],
                                      (((1,), (0,)), ((), ())),
                                      precision=jax.lax.Precision.DEFAULT)


def _shard_body(x, W1, b1, b2, W2, *, n, c, npad, bm, bmo, nshard):
    xpad = jnp.zeros((npad, c), x.dtype).at[:n].set(x)
    xt = xpad.T
    sqc = jnp.sum(xpad * xpad, axis=1)[None, :]
    sqc = jnp.where(jax.lax.iota(jnp.int32, npad)[None, :] >= n, 1e12, sqc)
    ab = jnp.concatenate([W1[:c] - W1[c:], W1[c:]], axis=1)

    nloc = npad // nshard
    d = jax.lax.axis_index("d")

    pq = pl.pallas_call(
        _pq_body,
        grid=(npad // 512,),
        in_specs=[pl.BlockSpec((512, c), lambda i: (i, 0)),
                  pl.BlockSpec((c, 2 * c), lambda i: (0, 0))],
        out_specs=pl.BlockSpec((512, 2 * c), lambda i: (i, 0)),
        out_shape=jax.ShapeDtypeStruct((npad, 2 * c), xpad.dtype),
        compiler_params=pltpu.CompilerParams(
            dimension_semantics=("parallel",)),
    )(xpad, ab)
    q_full = pq[:, c:]

    nsub = 2
    nsb = nloc // nsub
    outs = []
    idxs = []
    qgs = []
    for s in range(nsub):
        xs = jax.lax.dynamic_slice(xpad, (d * nloc + s * nsb, 0), (nsb, c))
        idx3 = pl.pallas_call(
            functools.partial(_knn_body, bm=bm, npad=npad, c=c),
            grid=(nsb // bm,),
            in_specs=[
                pl.BlockSpec((bm, c), lambda i: (i, 0)),
                pl.BlockSpec((c, npad), lambda i: (0, 0)),
                pl.BlockSpec((1, npad), lambda i: (0, 0)),
            ],
            out_specs=pl.BlockSpec((1, bm, K), lambda i: (i, 0, 0)),
            out_shape=jax.ShapeDtypeStruct((nsb // bm, bm, K), jnp.int32),
            scratch_shapes=[pltpu.VMEM((bm, npad), jnp.float32)],
            compiler_params=pltpu.CompilerParams(
                dimension_semantics=("parallel",)),
        )(xs, xt, sqc)
        idxs.append(idx3.reshape(nsb * K))
        qgs.append(_sc_gather(q_full, idxs[-1]))

    for s in range(nsub):
        p_loc = jax.lax.dynamic_slice(pq[:, :c], (d * nloc + s * nsb, 0),
                                      (nsb, c))
        out = pl.pallas_call(
            functools.partial(_mlp_body, bm=bmo, c=c),
            grid=(nsb // bmo,),
            in_specs=[
                pl.BlockSpec((bmo, c), lambda i: (i, 0)),
                pl.BlockSpec((bmo * K, c), lambda i: (i, 0)),
                pl.BlockSpec((c, c), lambda i: (0, 0)),
                pl.BlockSpec((1, c), lambda i: (0, 0)),
                pl.BlockSpec((1, c), lambda i: (0, 0)),
            ],
            out_specs=pl.BlockSpec((bmo, c), lambda i: (i, 0)),
            out_shape=jax.ShapeDtypeStruct((nsb, c), xpad.dtype),
            compiler_params=pltpu.CompilerParams(
                dimension_semantics=("parallel",)),
        )(p_loc, qgs[s], W2, b1[None, :], b2[None, :])
        outs.append(out)
    return jnp.concatenate(outs, axis=0)


def kernel(x, W1, b1, W2, b2):
    n, c = x.shape
    bm = 512
    bmo = 512
    npad = ((n + 2047) // 2048) * 2048

    ndev = len(jax.devices())
    nshard = 2 if (ndev >= 2 and npad % (2 * bm) == 0) else 1
    mesh = jax.make_mesh((nshard,), ("d",))
    P = jax.sharding.PartitionSpec
    body = functools.partial(_shard_body, n=n, c=c, npad=npad, bm=bm, bmo=bmo,
                             nshard=nshard)
    ns = functools.partial(jax.sharding.NamedSharding, mesh)
    rep = ns(P())
    args = [jax.reshard(a, rep) for a in (x, W1, b1, b2, W2)]
    out = jax.shard_map(
        body, mesh=mesh,
        in_specs=(P(), P(), P(), P(), P()),
        out_specs=P("d", None),
        check_vma=False,
    )(*args)
    return out[:n]


# speedup vs baseline: 1.0941x; 1.0309x over previous
import functools

import jax
import jax.numpy as jnp
from jax.experimental import pallas as pl
from jax.experimental.pallas import tpu as pltpu
from jax.experimental.pallas import tpu_sc as plsc

K = 16

def _sc_gather(table, idx_flat):
    num, c = idx_flat.shape[0], table.shape[1]
    window = 256
    assert num % window == 0
    idx2 = idx_flat.reshape(1, num)
    mesh = plsc.VectorSubcoreMesh(core_axis_name="core",
                                  subcore_axis_name="subcore")

    @functools.partial(
        pl.kernel,
        out_type=jax.ShapeDtypeStruct((num, c), table.dtype),
        mesh=mesh,
    )
    def gather_kernel(x_hbm, i_hbm, o_hbm):
        def body(i_vmem, o_vmem):
            pltpu.sync_copy(x_hbm.at[i_vmem.at[0]], o_vmem)

        pltpu.emit_pipeline(
            body,
            grid=(num // window,),
            in_specs=[pl.BlockSpec((1, window), index_map=lambda i: (0, i))],
            out_specs=[pl.BlockSpec((window, c), index_map=lambda i: (i, 0))],
            core_axis_name=("core", "subcore"),
            dimension_semantics=(pltpu.PARALLEL,),
        )(i_hbm, o_hbm)

    return gather_kernel(table, idx2)


def _knn_body(x_ref, xt_ref, sqc_ref, idx_ref, *, bm, npad, c):
    x = x_ref[...]
    sq_i = jnp.sum(x * x, axis=1, keepdims=True)

    lane = jax.lax.broadcasted_iota(jnp.int32, (bm, 128), 1)
    inf = jnp.float32(jnp.inf)
    m1 = m2 = m3 = m4 = jnp.full((bm, 128), inf, jnp.float32)
    i1 = i2 = i3 = i4 = jnp.full((bm, 128), npad, jnp.int32)
    grp = 2048
    for gi in range(npad // grp):
        dot = jax.lax.dot_general(
            x, xt_ref[:, gi * grp:(gi + 1) * grp], (((1,), (0,)), ((), ())),
            precision=jax.lax.Precision.DEFAULT)
        d2g = sq_i - 2.0 * dot + sqc_ref[:, gi * grp:(gi + 1) * grp]
        for rr in range(grp // 128):
            r = gi * (grp // 128) + rr
            v = d2g[:, rr * 128:(rr + 1) * 128]
            g = lane + r * 128
            c1 = v < m1
            c2 = v < m2
            c3 = v < m3
            c4 = v < m4
            m4 = jnp.where(c4, jnp.where(c3, m3, v), m4)
            i4 = jnp.where(c4, jnp.where(c3, i3, g), i4)
            m3 = jnp.where(c3, jnp.where(c2, m2, v), m3)
            i3 = jnp.where(c3, jnp.where(c2, i2, g), i3)
            m2 = jnp.where(c2, jnp.where(c1, m1, v), m2)
            i2 = jnp.where(c2, jnp.where(c1, i1, g), i2)
            m1 = jnp.where(c1, v, m1)
            i1 = jnp.where(c1, g, i1)
    cv = jnp.concatenate([m1, m2, m3, m4], axis=1)
    ci = jnp.concatenate([i1, i2, i3, i4], axis=1)
    cols = []
    for _ in range(K):
        m = jnp.min(cv, axis=1, keepdims=True)
        j = jnp.min(jnp.where(cv == m, ci, npad), axis=1, keepdims=True)
        cols.append(j)
        cv = jnp.where(ci == j, inf, cv)
    idx_ref[...] = jnp.concatenate(cols, axis=1)[None]


def _mlp_body(p_ref, qg_ref, w2_ref, b1_ref, b2_ref, o_ref, *, bm, c):
    p = p_ref[...]
    qg = qg_ref[...].reshape(bm, K, c)
    h1 = jnp.maximum(qg + p[:, None, :] + b1_ref[...], 0.0)
    h2 = jax.lax.dot_general(h1.reshape(bm * K, c), w2_ref[...],
                             (((1,), (0,)), ((), ())),
                             precision=jax.lax.Precision.DEFAULT)
    h2 = jnp.maximum(h2 + b2_ref[...], 0.0)
    o_ref[...] = jnp.mean(h2.reshape(bm, K, c), axis=1)


def _pq_body(x_ref, ab_ref, pq_ref):
    pq_ref[...] = jax.lax.dot_general(x_ref[...], ab_ref[...],
                                      (((1,), (0,)), ((), ())),
                                      precision=jax.lax.Precision.DEFAULT)


def _shard_body(x, W1, b1, b2, W2, *, n, c, npad, bm, bmo, nshard):
    xpad = jnp.zeros((npad, c), x.dtype).at[:n].set(x)
    xt = xpad.T
    sqc = jnp.sum(xpad * xpad, axis=1)[None, :]
    sqc = jnp.where(jax.lax.iota(jnp.int32, npad)[None, :] >= n, 1e12, sqc)
    ab = jnp.concatenate([W1[:c] - W1[c:], W1[c:]], axis=1)

    nloc = npad // nshard
    d = jax.lax.axis_index("d")

    pq = pl.pallas_call(
        _pq_body,
        grid=(npad // 512,),
        in_specs=[pl.BlockSpec((512, c), lambda i: (i, 0)),
                  pl.BlockSpec((c, 2 * c), lambda i: (0, 0))],
        out_specs=pl.BlockSpec((512, 2 * c), lambda i: (i, 0)),
        out_shape=jax.ShapeDtypeStruct((npad, 2 * c), xpad.dtype),
        compiler_params=pltpu.CompilerParams(
            dimension_semantics=("parallel",)),
    )(xpad, ab)
    q_full = pq[:, c:]

    nsub = 2
    nsb = nloc // nsub
    outs = []
    idxs = []
    qgs = []
    for s in range(nsub):
        xs = jax.lax.dynamic_slice(xpad, (d * nloc + s * nsb, 0), (nsb, c))
        idx3 = pl.pallas_call(
            functools.partial(_knn_body, bm=bm, npad=npad, c=c),
            grid=(nsb // bm,),
            in_specs=[
                pl.BlockSpec((bm, c), lambda i: (i, 0)),
                pl.BlockSpec((c, npad), lambda i: (0, 0)),
                pl.BlockSpec((1, npad), lambda i: (0, 0)),
            ],
            out_specs=pl.BlockSpec((1, bm, K), lambda i: (i, 0, 0)),
            out_shape=jax.ShapeDtypeStruct((nsb // bm, bm, K), jnp.int32),
            compiler_params=pltpu.CompilerParams(
                dimension_semantics=("parallel",)),
        )(xs, xt, sqc)
        idxs.append(idx3.reshape(nsb * K))
        qgs.append(_sc_gather(q_full, idxs[-1]))

    for s in range(nsub):
        p_loc = jax.lax.dynamic_slice(pq[:, :c], (d * nloc + s * nsb, 0),
                                      (nsb, c))
        out = pl.pallas_call(
            functools.partial(_mlp_body, bm=bmo, c=c),
            grid=(nsb // bmo,),
            in_specs=[
                pl.BlockSpec((bmo, c), lambda i: (i, 0)),
                pl.BlockSpec((bmo * K, c), lambda i: (i, 0)),
                pl.BlockSpec((c, c), lambda i: (0, 0)),
                pl.BlockSpec((1, c), lambda i: (0, 0)),
                pl.BlockSpec((1, c), lambda i: (0, 0)),
            ],
            out_specs=pl.BlockSpec((bmo, c), lambda i: (i, 0)),
            out_shape=jax.ShapeDtypeStruct((nsb, c), xpad.dtype),
            compiler_params=pltpu.CompilerParams(
                dimension_semantics=("parallel",)),
        )(p_loc, qgs[s], W2, b1[None, :], b2[None, :])
        outs.append(out)
    return jnp.concatenate(outs, axis=0)


def kernel(x, W1, b1, W2, b2):
    n, c = x.shape
    bm = 512
    bmo = 512
    npad = ((n + 2047) // 2048) * 2048

    ndev = len(jax.devices())
    nshard = 2 if (ndev >= 2 and npad % (2 * bm) == 0) else 1
    mesh = jax.make_mesh((nshard,), ("d",))
    P = jax.sharding.PartitionSpec
    body = functools.partial(_shard_body, n=n, c=c, npad=npad, bm=bm, bmo=bmo,
                             nshard=nshard)
    ns = functools.partial(jax.sharding.NamedSharding, mesh)
    rep = ns(P())
    args = [jax.reshard(a, rep) for a in (x, W1, b1, b2, W2)]
    out = jax.shard_map(
        body, mesh=mesh,
        in_specs=(P(), P(), P(), P(), P()),
        out_specs=P("d", None),
        check_vma=False,
    )(*args)
    return out[:n]
